# baseline (speedup 1.0000x reference)
import functools

import jax
import jax.numpy as jnp
from jax import lax
from jax.experimental import pallas as pl
from jax.experimental.pallas import tpu as pltpu
from jax.experimental.pallas import tpu_sc as plsc

_VOCAB = 22
_HIDDEN = 2048
_EPS = 1e-12

_B = 4 * 2048
_NC, _NS = 2, 16
_NW = _NC * _NS
_BPW = _B // _NW
_LANES = 16
_K = 64


def _norm_body(tab_ref, gamma_ref, beta_ref, out_ref):
    t = tab_ref[...]
    mean = jnp.mean(t, axis=1, keepdims=True)
    c = t - mean
    var = jnp.mean(c * c, axis=1, keepdims=True)
    out_ref[...] = c * lax.rsqrt(var + _EPS) * gamma_ref[...] + beta_ref[...]


def _normalize_table(table, gamma, beta):
    return pl.pallas_call(
        _norm_body,
        out_shape=jax.ShapeDtypeStruct((_VOCAB, _HIDDEN), jnp.float32),
    )(table, gamma.reshape(1, _HIDDEN), beta.reshape(1, _HIDDEN))


_sc_mesh = plsc.VectorSubcoreMesh(core_axis_name="c", subcore_axis_name="s")


@functools.partial(
    pl.kernel,
    mesh=_sc_mesh,
    out_type=jax.ShapeDtypeStruct((_B, _HIDDEN), jnp.float32),
    scratch_types=(
        [pltpu.VMEM((_BPW,), jnp.int32),
         pltpu.VMEM((_VOCAB, _HIDDEN), jnp.float32)]
        + [pltpu.SemaphoreType.DMA, pltpu.SemaphoreType.DMA]
    ),
)
def _sc_scatter_rows(idx_hbm, tab_hbm, out_hbm, idx_v, tab_v, sem, ssem):
    wid = lax.axis_index("s") * _NC + lax.axis_index("c")
    base = wid * _BPW
    stage_idx = pltpu.async_copy(idx_hbm.at[pl.ds(base, _BPW)], idx_v, ssem)
    stage_tab = pltpu.async_copy(tab_hbm, tab_v, ssem)
    stage_idx.wait()
    stage_tab.wait()

    def fire(g):
        descs = []
        for half in range(_K // _LANES):
            iv = idx_v[pl.ds(g * _K + half * _LANES, _LANES)]
            for b in range(_LANES):
                r = g * _K + half * _LANES + b
                descs.append(pltpu.async_copy(
                    tab_v.at[pl.ds(iv[b], 1)],
                    out_hbm.at[pl.ds(base + r, 1)],
                    sem))
        return descs

    def group(g, carry):
        for d in fire(g):
            d.wait()
        return carry

    lax.fori_loop(0, _BPW // _K, group, 0)


def kernel(seq, table, gamma, beta):
    norm_tab = _normalize_table(table, gamma, beta)
    idx = seq.reshape(-1).astype(jnp.int32)
    out = _sc_scatter_rows(idx, norm_tab)
    return out.reshape(seq.shape[0], seq.shape[1], _HIDDEN)


# speedup vs baseline: 1.1184x; 1.1184x over previous
import functools

import jax
import jax.numpy as jnp
from jax import lax
from jax.experimental import pallas as pl
from jax.experimental.pallas import tpu as pltpu
from jax.experimental.pallas import tpu_sc as plsc

_VOCAB = 22
_HIDDEN = 2048
_EPS = 1e-12

_B = 4 * 2048
_NC, _NS = 2, 16
_NW = _NC * _NS
_BPW = _B // _NW
_LANES = 16
_K = 16


def _norm_body(tab_ref, gamma_ref, beta_ref, out_ref):
    t = tab_ref[...]
    mean = jnp.mean(t, axis=1, keepdims=True)
    c = t - mean
    var = jnp.mean(c * c, axis=1, keepdims=True)
    out_ref[...] = c * lax.rsqrt(var + _EPS) * gamma_ref[...] + beta_ref[...]


def _normalize_table(table, gamma, beta):
    return pl.pallas_call(
        _norm_body,
        out_shape=jax.ShapeDtypeStruct((_VOCAB, _HIDDEN), jnp.float32),
    )(table, gamma.reshape(1, _HIDDEN), beta.reshape(1, _HIDDEN))


_sc_mesh = plsc.VectorSubcoreMesh(core_axis_name="c", subcore_axis_name="s")


@functools.partial(
    pl.kernel,
    mesh=_sc_mesh,
    out_type=jax.ShapeDtypeStruct((_B, _HIDDEN), jnp.float32),
    scratch_types=(
        [pltpu.VMEM((_BPW,), jnp.int32),
         pltpu.VMEM((_VOCAB, _HIDDEN), jnp.float32)]
        + [pltpu.SemaphoreType.DMA, pltpu.SemaphoreType.DMA]
    ),
)
def _sc_scatter_rows(idx_hbm, tab_hbm, out_hbm, idx_v, tab_v, sem, ssem):
    wid = lax.axis_index("s") * _NC + lax.axis_index("c")
    base = wid * _BPW
    stage_idx = pltpu.async_copy(idx_hbm.at[pl.ds(base, _BPW)], idx_v, ssem)
    stage_tab = pltpu.async_copy(tab_hbm, tab_v, ssem)
    stage_idx.wait()
    stage_tab.wait()

    def fire(g):
        descs = []
        for half in range(_K // _LANES):
            iv = idx_v[pl.ds(g * _K + half * _LANES, _LANES)]
            for b in range(_LANES):
                r = g * _K + half * _LANES + b
                descs.append(pltpu.async_copy(
                    tab_v.at[pl.ds(iv[b], 1)],
                    out_hbm.at[pl.ds(base + r, 1)],
                    sem))
        return descs

    def group(g, carry):
        for d in fire(g):
            d.wait()
        return carry

    lax.fori_loop(0, _BPW // _K, group, 0)


def kernel(seq, table, gamma, beta):
    norm_tab = _normalize_table(table, gamma, beta)
    idx = seq.reshape(-1).astype(jnp.int32)
    out = _sc_scatter_rows(idx, norm_tab)
    return out.reshape(seq.shape[0], seq.shape[1], _HIDDEN)
